# baseline (speedup 1.0000x reference)
import functools

import jax
import jax.numpy as jnp
from jax import lax
from jax.experimental import pallas as pl
from jax.experimental.pallas import tpu as pltpu
from jax.experimental.pallas import tpu_sc as plsc

_B = 16384
_C = 1000
_N = 1000
_NC = 2
_NS = 16
_NW = _NC * _NS
_BPW = _B // _NW
_R = 32
_NCH = _BPW // _R
_SPR = 1024

_mesh = plsc.VectorSubcoreMesh(core_axis_name="c", subcore_axis_name="s")


@functools.partial(
    pl.kernel,
    out_type=(
        jax.ShapeDtypeStruct((_NC, _SPR, _N), jnp.float32),
        jax.ShapeDtypeStruct((_NC, _SPR, 16), jnp.float32),
        jax.ShapeDtypeStruct((_NW, 16), jnp.float32),
    ),
    mesh=_mesh,
    compiler_params=pltpu.CompilerParams(use_tc_tiling_on_sc=False),
    scratch_types=[
        pltpu.VMEM((_NCH, _R), jnp.int32),
        pltpu.VMEM((_R, _N), jnp.float32),
        pltpu.VMEM((_R, 16), jnp.float32),
        pltpu.VMEM((64, 16), jnp.float32),
        pltpu.VMEM((16,), jnp.float32),
        pltpu.VMEM_SHARED((_SPR, _N), jnp.float32),
        pltpu.VMEM_SHARED((_SPR, 16), jnp.float32),
    ],
)
def _sc_segsum(x_hbm, y_hbm, s_out, cnt_out, xsq_out,
               idx_v, xc_v, ones_v, zc_v, acc_v, s_sh, cnt_sh):
    c = lax.axis_index("c")
    s = lax.axis_index("s")
    wid = c * _NS + s
    base = wid * _BPW
    zeros16 = jnp.zeros((16,), jnp.float32)

    def ld_idx(j, carry):
        pltpu.sync_copy(y_hbm.at[pl.ds(base + j * _R, _R)], idx_v.at[j])
        return carry
    lax.fori_loop(0, _NCH, ld_idx, 0)

    def zrow(r, carry):
        def zcol(ci, carry2):
            xc_v[r, pl.ds(ci * 16, 16)] = zeros16
            return carry2
        lax.fori_loop(0, 62, zcol, 0)
        xc_v[r, pl.ds(_N - 16, 16)] = zeros16
        return carry
    lax.fori_loop(0, _R, zrow, 0)
    pltpu.sync_copy(xc_v, s_sh.at[pl.ds(s * 64, _R)])
    pltpu.sync_copy(xc_v, s_sh.at[pl.ds(s * 64 + _R, _R)])

    def zcnt(r, carry):
        zc_v[r, :] = zeros16
        return carry
    lax.fori_loop(0, 64, zcnt, 0)
    pltpu.sync_copy(zc_v, cnt_sh.at[pl.ds(s * 64, 64)])

    def fill_ones(r, carry):
        ones_v[r, :] = jnp.ones((16,), jnp.float32)
        return carry
    lax.fori_loop(0, _R, fill_ones, 0)

    plsc.subcore_barrier()

    lane = lax.broadcasted_iota(jnp.int32, (16,), 0)

    def chunk(j, acc):
        pltpu.sync_copy(x_hbm.at[pl.ds(base + j * _R, _R)], xc_v)
        pltpu.sync_copy(xc_v, s_sh.at[idx_v.at[j]], add=True)
        pltpu.sync_copy(ones_v, cnt_sh.at[idx_v.at[j]], add=True)

        def row(r, acc_r):
            def col(ci, acc_c):
                v = xc_v[r, pl.ds(ci * 16, 16)]
                return acc_c + v * v
            acc_r = lax.fori_loop(0, 62, col, acc_r, unroll=8)
            tail = xc_v[r, pl.ds(_N - 16, 16)]
            tail = jnp.where(lane >= 8, tail, 0.0)
            return acc_r + tail * tail
        return lax.fori_loop(0, _R, row, acc)

    acc = lax.fori_loop(0, _NCH, chunk, jnp.zeros((16,), jnp.float32))
    acc_v[...] = acc
    pltpu.sync_copy(acc_v, xsq_out.at[wid])

    plsc.subcore_barrier()

    pltpu.sync_copy(s_sh.at[pl.ds(s * 64, 64)], s_out.at[c, pl.ds(s * 64, 64)])
    pltpu.sync_copy(cnt_sh.at[pl.ds(s * 64, 64)],
                    cnt_out.at[c, pl.ds(s * 64, 64)])


def _combine_kernel(s_ref, cnt_ref, xsq_ref, centers_ref, out_ref):
    ctr = centers_ref[...]
    ssum = s_ref[0, 0:_C, :] + s_ref[1, 0:_C, :]
    t2 = jnp.sum(ssum * ctr)
    n = jnp.sum(cnt_ref[0, 0:_C, :] + cnt_ref[1, 0:_C, :],
                axis=1, keepdims=True)
    rn = jnp.sum(ctr * ctr, axis=1, keepdims=True)
    t3 = jnp.sum(n * rn)
    sumxsq = jnp.sum(xsq_ref[...])
    out_ref[0, 0] = (sumxsq - 2.0 * t2 + t3) * (1.0 / (_B * _N))


def kernel(x, y, centers):
    y32 = y.astype(jnp.int32)
    s, cnt, xsq = _sc_segsum(x, y32)
    total = pl.pallas_call(
        _combine_kernel,
        out_specs=pl.BlockSpec(memory_space=pltpu.SMEM),
        out_shape=jax.ShapeDtypeStruct((1, 1), jnp.float32),
    )(s, cnt, xsq, centers)
    return total[0, 0]


# speedup vs baseline: 1.8699x; 1.8699x over previous
import functools

import jax
import jax.numpy as jnp
from jax import lax
from jax.experimental import pallas as pl
from jax.experimental.pallas import tpu as pltpu
from jax.experimental.pallas import tpu_sc as plsc

_B = 16384
_C = 1000
_N = 1000
_NP = 1024
_NC = 2
_NS = 16
_NW = _NC * _NS
_BPW = _B // _NW
_R = 16
_NCH = _BPW // _R

_mesh = plsc.VectorSubcoreMesh(core_axis_name="c", subcore_axis_name="s")


@functools.partial(
    pl.kernel,
    out_type=jax.ShapeDtypeStruct((_NW, 16), jnp.float32),
    mesh=_mesh,
    scratch_types=[
        pltpu.VMEM((_NCH, _R), jnp.int32),
        pltpu.VMEM((_R, _N), jnp.float32),
        pltpu.VMEM((_R, _N), jnp.float32),
        pltpu.VMEM((_R, _NP), jnp.float32),
        pltpu.VMEM((_R, _NP), jnp.float32),
        pltpu.VMEM((16,), jnp.float32),
        pltpu.SemaphoreType.DMA,
        pltpu.SemaphoreType.DMA,
        pltpu.SemaphoreType.DMA,
        pltpu.SemaphoreType.DMA,
    ],
)
def _sc_mse(x_hbm, y_hbm, ctr_hbm, out_hbm,
            idx_v, xc0, xc1, g0, g1, acc_v, si0, si1, sg0, sg1):
    xc = (xc0, xc1)
    g = (g0, g1)
    sem_in = (si0, si1)
    sem_g = (sg0, sg1)

    c = lax.axis_index("c")
    s = lax.axis_index("s")
    wid = c * _NS + s
    base = wid * _BPW
    lane = lax.broadcasted_iota(jnp.int32, (16,), 0)

    def ld_idx(j, carry):
        pltpu.sync_copy(y_hbm.at[pl.ds(base + j * _R, _R)], idx_v.at[j])
        return carry
    lax.fori_loop(0, _NCH, ld_idx, 0)

    acc_v[...] = jnp.zeros((16,), jnp.float32)

    for b in range(2):
        pltpu.async_copy(x_hbm.at[pl.ds(base + b * _R, _R)], xc[b], sem_in[b])
        pltpu.async_copy(ctr_hbm.at[idx_v.at[b]], g[b], sem_g[b])

    def pair(jj, carry):
        for b in range(2):
            j = jj * 2 + b
            pltpu.make_async_copy(
                x_hbm.at[pl.ds(base + j * _R, _R)], xc[b], sem_in[b]).wait()
            pltpu.make_async_copy(
                ctr_hbm.at[idx_v.at[j]], g[b], sem_g[b]).wait()

            def row(r, acc_r):
                def col(ci, acc_c):
                    vx = xc[b][r, pl.ds(ci * 16, 16)]
                    vg = g[b][r, pl.ds(ci * 16, 16)]
                    d = vx - vg
                    return acc_c + d * d
                acc_r = lax.fori_loop(0, 62, col, acc_r, unroll=8)
                tx = xc[b][r, pl.ds(_N - 16, 16)]
                tg = g[b][r, pl.ds(_N - 16, 16)]
                d = jnp.where(lane >= 8, tx - tg, 0.0)
                return acc_r + d * d
            acc = lax.fori_loop(0, _R, row, acc_v[...])
            acc_v[...] = acc

            @pl.when(j < _NCH - 2)
            def _prefetch():
                pltpu.async_copy(
                    x_hbm.at[pl.ds(base + (j + 2) * _R, _R)], xc[b],
                    sem_in[b])
                pltpu.async_copy(
                    ctr_hbm.at[idx_v.at[j + 2]], g[b], sem_g[b])
        return carry

    lax.fori_loop(0, _NCH // 2, pair, 0)

    pltpu.sync_copy(acc_v, out_hbm.at[wid])


def kernel(x, y, centers):
    y32 = y.astype(jnp.int32)
    ctr_p = jnp.pad(centers, ((0, 0), (0, _NP - _N)))
    partials = _sc_mse(x, y32, ctr_p)
    return (jnp.sum(partials) * (1.0 / (_B * _N))).astype(jnp.float32)
